# Initial kernel scaffold; baseline (speedup 1.0000x reference)
import jax
import jax.numpy as jnp
from jax.experimental import pallas as pl


def kernel(x_lex, x_morph, edge_index, batch, prototypes, params):
    raise NotImplementedError("write your pallas kernel here")


# speedup vs baseline: 1.1613x; 1.1613x over previous
import functools

import jax
import jax.numpy as jnp
from jax import lax
from jax.experimental import pallas as pl
from jax.experimental.pallas import tpu as pltpu
from jax.experimental.pallas import tpu_sc as plsc

_N = 50000
_E = 800000
_ED = 32
_HD = 64
_ND = 64
_B = 64
_NC = 2
_NS = 16
_L = 16
_NW = _NC * _NS
_NP = 50176
_EP = 819200
_NPT = _NP // _NW
_EPT = _EP // _NW
_CHUNK = 512
_JROWS = _CHUNK // 128
_NCHUNK = _EPT // _CHUNK
_DEGW = 16
_SPROWS = _NP // _NS

_f32 = jnp.float32
_i32 = jnp.int32

_MESH = plsc.VectorSubcoreMesh(core_axis_name="c", subcore_axis_name="s")
_SC_PARAMS = pltpu.CompilerParams(use_tc_tiling_on_sc=False,
                                  needs_layout_passes=False)


def _wid():
    return lax.axis_index("s") * _NC + lax.axis_index("c")


def _sc_embed_body(emb_hbm, lex_hbm, out_hbm, idx_v, rows_v, sem):
    base = _wid() * _NPT
    pltpu.sync_copy(lex_hbm.at[pl.ds(base, _NPT)], idx_v)
    cps = []
    for i in range(14):
        o = i * 112
        cps.append(pltpu.async_copy(
            emb_hbm.at[idx_v.at[pl.ds(o, 112)]], rows_v.at[pl.ds(o, 112)], sem))
    for cp in cps:
        cp.wait()
    pltpu.sync_copy(rows_v, out_hbm.at[pl.ds(base, _NPT)])


_sc_embed = pl.kernel(
    _sc_embed_body,
    out_type=jax.ShapeDtypeStruct((_NP, _ED), _f32),
    mesh=_MESH,
    compiler_params=_SC_PARAMS,
    scratch_types=[
        pltpu.VMEM((_NPT,), _i32),
        pltpu.VMEM((_NPT, _ED), _f32),
        pltpu.SemaphoreType.DMA,
    ],
)


def _sc_mask_body(u_hbm, v_hbm, src_hbm, dst_hbm, wb_hbm, zdeg_hbm,
                  m_hbm, deg_hbm,
                  idx_s, idx_d, ubuf, vbuf, mbuf, dbuf, wbv, z16, deg_sp, sem):
    c = lax.axis_index("c")
    s = lax.axis_index("s")
    w = _wid()
    pltpu.sync_copy(zdeg_hbm.at[pl.ds(s * _SPROWS, _SPROWS)],
                    deg_sp.at[pl.ds(s * _SPROWS, _SPROWS)])
    pltpu.sync_copy(zdeg_hbm.at[pl.ds(0, _CHUNK)], dbuf)
    pltpu.sync_copy(wb_hbm, wbv)
    pltpu.sync_copy(zdeg_hbm.at[0].at[pl.ds(0, _L)], z16)
    plsc.subcore_barrier()

    iota = lax.iota(_i32, _L)
    zero16 = plsc.bitcast(z16[...], _i32)
    one16 = jnp.full((_L,), 1, _i32)
    bv2 = plsc.load_gather(wbv, [one16, zero16])

    def chunk_body(ch, carry):
        ebase = w * _EPT + ch * _CHUNK
        rbase = w * (_EPT // 128) + ch * _JROWS
        pltpu.sync_copy(src_hbm.at[pl.ds(rbase, _JROWS)], idx_s)
        pltpu.sync_copy(dst_hbm.at[pl.ds(rbase, _JROWS)], idx_d)
        cps = []
        for j in range(_JROWS):
            cps.append(pltpu.async_copy(
                u_hbm.at[idx_s.at[j]], ubuf.at[pl.ds(j * 128, 128)], sem))
            cps.append(pltpu.async_copy(
                v_hbm.at[idx_d.at[j]], vbuf.at[pl.ds(j * 128, 128)], sem))
        for cp in cps:
            cp.wait()
        plsc.subcore_barrier()

        def g_body(g, carry2):
            eidx = iota + g * _L
            acc = bv2
            for f in range(_ND):
                f16 = zero16 if f == 0 else jnp.full((_L,), f, _i32)
                uf = plsc.load_gather(ubuf, [eidx, f16])
                vf = plsc.load_gather(vbuf, [eidx, f16])
                w2f = plsc.load_gather(wbv, [zero16, f16])
                acc = acc + jnp.maximum(uf + vf, 0.0) * w2f
            m16 = 1.0 / (1.0 + jnp.exp(-acc))
            geid = ebase + g * _L + iota
            m16 = jnp.where(geid < _E, m16, 0.0)
            mbuf[pl.ds(g * _L, _L)] = m16
            plsc.store_scatter(dbuf, [eidx, zero16], m16)
            return carry2

        lax.fori_loop(0, _CHUNK // _L, g_body, 0, unroll=False)
        pltpu.sync_copy(mbuf, m_hbm.at[pl.ds(ebase, _CHUNK)])
        for j in range(_JROWS):
            pltpu.sync_copy(dbuf.at[pl.ds(j * 128, 128)],
                            deg_sp.at[idx_d.at[j]], add=True)
        return carry

    lax.fori_loop(0, _NCHUNK, chunk_body, 0, unroll=False)
    plsc.subcore_barrier()
    pltpu.sync_copy(deg_sp.at[pl.ds(s * _SPROWS, _SPROWS)],
                    deg_hbm.at[c].at[pl.ds(s * _SPROWS, _SPROWS)])


_sc_mask = pl.kernel(
    _sc_mask_body,
    out_type=(jax.ShapeDtypeStruct((_EP,), _f32),
              jax.ShapeDtypeStruct((_NC, _NP, _DEGW), _f32)),
    mesh=_MESH,
    compiler_params=_SC_PARAMS,
    scratch_types=[
        pltpu.VMEM((_JROWS, 128), _i32),
        pltpu.VMEM((_JROWS, 128), _i32),
        pltpu.VMEM((_CHUNK, _ND), _f32),
        pltpu.VMEM((_CHUNK, _ND), _f32),
        pltpu.VMEM((_CHUNK,), _f32),
        pltpu.VMEM((_CHUNK, _DEGW), _f32),
        pltpu.VMEM((2, _ND), _f32),
        pltpu.VMEM((_L,), _f32),
        pltpu.VMEM_SHARED((_NP, _DEGW), _f32),
        pltpu.SemaphoreType.DMA,
    ],
)


def _sc_agg_body(tab_hbm, m_hbm, src_hbm, dst_hbm, zacc_hbm,
                 acc_out_hbm,
                 idx_s, idx_d, tbuf, mbuf, z16, acc_sp, sem):
    c = lax.axis_index("c")
    s = lax.axis_index("s")
    pltpu.sync_copy(zacc_hbm.at[pl.ds(s * _SPROWS, _SPROWS)],
                    acc_sp.at[pl.ds(s * _SPROWS, _SPROWS)])
    pltpu.sync_copy(zacc_hbm.at[0].at[pl.ds(0, _L)], z16)
    plsc.subcore_barrier()

    iota = lax.iota(_i32, _L)
    zero16 = plsc.bitcast(z16[...], _i32)
    _EPS = _EP // _NS

    def chunk_body(ch, carry):
        ebase = s * _EPS + ch * _CHUNK
        rbase = s * (_EPS // 128) + ch * _JROWS
        pltpu.sync_copy(src_hbm.at[pl.ds(rbase, _JROWS)], idx_s)
        pltpu.sync_copy(dst_hbm.at[pl.ds(rbase, _JROWS)], idx_d)
        pltpu.sync_copy(m_hbm.at[pl.ds(ebase, _CHUNK)], mbuf)
        cps = []
        for j in range(_JROWS):
            cps.append(pltpu.async_copy(
                tab_hbm.at[c].at[idx_s.at[j]], tbuf.at[pl.ds(j * 128, 128)],
                sem))
        for cp in cps:
            cp.wait()
        plsc.subcore_barrier()

        def g_body(g, carry2):
            eidx = iota + g * _L
            m16 = mbuf[pl.ds(g * _L, _L)]
            for f in range(_ED):
                f16 = zero16 if f == 0 else jnp.full((_L,), f, _i32)
                col = plsc.load_gather(tbuf, [eidx, f16]) * m16
                plsc.store_scatter(tbuf, [eidx, f16], col)
            return carry2

        lax.fori_loop(0, _CHUNK // _L, g_body, 0, unroll=False)
        for j in range(_JROWS):
            pltpu.sync_copy(tbuf.at[pl.ds(j * 128, 128)],
                            acc_sp.at[idx_d.at[j]], add=True)
        return carry

    lax.fori_loop(0, _EPS // _CHUNK, chunk_body, 0, unroll=False)
    plsc.subcore_barrier()
    pltpu.sync_copy(acc_sp.at[pl.ds(s * _SPROWS, _SPROWS)],
                    acc_out_hbm.at[c].at[pl.ds(s * _SPROWS, _SPROWS)])


_sc_agg = pl.kernel(
    _sc_agg_body,
    out_type=jax.ShapeDtypeStruct((_NC, _NP, _ED), _f32),
    mesh=_MESH,
    compiler_params=_SC_PARAMS,
    scratch_types=[
        pltpu.VMEM((_JROWS, 128), _i32),
        pltpu.VMEM((_JROWS, 128), _i32),
        pltpu.VMEM((_CHUNK, _ED), _f32),
        pltpu.VMEM((_CHUNK,), _f32),
        pltpu.VMEM((_L,), _f32),
        pltpu.VMEM_SHARED((_NP, _ED), _f32),
        pltpu.SemaphoreType.DMA,
    ],
)


_R = 1568
_G = _NP // _R


def _dot(a, b):
    return jnp.dot(a, b, preferred_element_type=_f32)


def _tc_prep_body(xle, xmorph, wm, bm, wvs, wvd, bv1, wnbr,
                  x_out, xm_out, u_out, v_out, t1_out):
    xm = jnp.maximum(_dot(xmorph[...], wm[...]) + bm[...], 0.0)
    x = jnp.concatenate([xle[...], xm], axis=1)
    x_out[...] = x
    xm_out[...] = xm
    u_out[...] = _dot(x, wvs[...]) + bv1[...]
    v_out[...] = _dot(x, wvd[...])
    t1 = _dot(x, wnbr[...])
    t1_out[0] = t1[:, :_ED]
    t1_out[1] = t1[:, _ED:]


def _full(shape):
    return pl.BlockSpec(shape, lambda i: (0,) * len(shape))


_tc_prep = pl.pallas_call(
    _tc_prep_body,
    grid=(_G,),
    in_specs=[
        pl.BlockSpec((_R, _ED), lambda i: (i, 0)),
        pl.BlockSpec((_R, 16), lambda i: (i, 0)),
        _full((16, _ED)), _full((1, _ED)),
        _full((_ND, _HD)), _full((_ND, _HD)), _full((1, _HD)),
        _full((_ND, _HD)),
    ],
    out_specs=[
        pl.BlockSpec((_R, _ND), lambda i: (i, 0)),
        pl.BlockSpec((_R, _ED), lambda i: (i, 0)),
        pl.BlockSpec((_R, _ND), lambda i: (i, 0)),
        pl.BlockSpec((_R, _ND), lambda i: (i, 0)),
        pl.BlockSpec((2, _R, _ED), lambda i: (0, i, 0)),
    ],
    out_shape=[
        jax.ShapeDtypeStruct((_NP, _ND), _f32),
        jax.ShapeDtypeStruct((_NP, _ED), _f32),
        jax.ShapeDtypeStruct((_NP, _ND), _f32),
        jax.ShapeDtypeStruct((_NP, _ND), _f32),
        jax.ShapeDtypeStruct((2, _NP, _ED), _f32),
    ],
)


def _node_update(h, acc, deg2, xm, p8, wself, wq, wctx, wmrf, b):
    deg = deg2[0][:, 0:1] + deg2[1][:, 0:1]
    agg = jnp.concatenate([acc[0], acc[1]], axis=1) / (deg + 1e-6)
    q = _dot(h, wq)
    scores = lax.dot_general(q, p8, (((1,), (1,)), ((), ())),
                             preferred_element_type=_f32)
    col = lax.broadcasted_iota(_i32, (1, 8), 1)
    scores = jnp.where(col < 5, scores, -1e30)
    scores = scores - jnp.max(scores, axis=1, keepdims=True)
    e = jnp.exp(scores)
    alpha = e / jnp.sum(e, axis=1, keepdims=True)
    ctx = _dot(alpha, p8)
    return jnp.maximum(
        _dot(h, wself) + agg + _dot(ctx, wctx) + _dot(xm, wmrf) + b[...], 0.0)


def _tc_layer1_body(h, acc, deg2, xm, p8, wself, wq, wctx, wmrf, b, wnbr2,
                    h1_out, t2_out):
    h1 = _node_update(h[...], acc[...], deg2[...], xm[...], p8[...],
                      wself[...], wq[...], wctx[...], wmrf[...], b)
    h1_out[...] = h1
    t2 = _dot(h1, wnbr2[...])
    t2_out[0] = t2[:, :_ED]
    t2_out[1] = t2[:, _ED:]


_tc_layer1 = pl.pallas_call(
    _tc_layer1_body,
    grid=(_G,),
    in_specs=[
        pl.BlockSpec((_R, _ND), lambda i: (i, 0)),
        pl.BlockSpec((2, _R, _ED), lambda i: (0, i, 0)),
        pl.BlockSpec((2, _R, _DEGW), lambda i: (0, i, 0)),
        pl.BlockSpec((_R, _ED), lambda i: (i, 0)),
        _full((8, _HD)),
        _full((_ND, _HD)), _full((_ND, _HD)), _full((_HD, _HD)),
        _full((_ED, _HD)), _full((1, _HD)),
        _full((_HD, _HD)),
    ],
    out_specs=[
        pl.BlockSpec((_R, _HD), lambda i: (i, 0)),
        pl.BlockSpec((2, _R, _ED), lambda i: (0, i, 0)),
    ],
    out_shape=[
        jax.ShapeDtypeStruct((_NP, _HD), _f32),
        jax.ShapeDtypeStruct((2, _NP, _ED), _f32),
    ],
)


def _tc_layer2_body(h, acc, deg2, xm, p8, wself, wq, wctx, wmrf, b,
                    bat, wc1, bc1, wc2, bc2,
                    logits_out, ge_out, ge_acc, cnt_acc):
    i = pl.program_id(0)
    h2 = _node_update(h[...], acc[...], deg2[...], xm[...], p8[...],
                      wself[...], wq[...], wctx[...], wmrf[...], b)
    onehot = (bat[...] == lax.broadcasted_iota(_i32, (1, _B), 1)).astype(_f32)

    @pl.when(i == 0)
    def _():
        ge_acc[...] = jnp.zeros_like(ge_acc)
        cnt_acc[...] = jnp.zeros_like(cnt_acc)

    ge_acc[...] += lax.dot_general(onehot, h2, (((0,), (0,)), ((), ())),
                                   preferred_element_type=_f32)
    cnt_acc[...] += lax.dot_general(
        onehot, jnp.ones((_R, 1), _f32), (((0,), (0,)), ((), ())),
        preferred_element_type=_f32)

    @pl.when(i == _G - 1)
    def _():
        ge = ge_acc[...] / (cnt_acc[...] + 1e-6)
        hid = jnp.maximum(_dot(ge, wc1[...]) + bc1[...], 0.0)
        logits_out[...] = _dot(hid, wc2[...]) + bc2[...]
        ge_out[...] = ge


_tc_layer2 = pl.pallas_call(
    _tc_layer2_body,
    grid=(_G,),
    in_specs=[
        pl.BlockSpec((_R, _HD), lambda i: (i, 0)),
        pl.BlockSpec((2, _R, _ED), lambda i: (0, i, 0)),
        pl.BlockSpec((2, _R, _DEGW), lambda i: (0, i, 0)),
        pl.BlockSpec((_R, _ED), lambda i: (i, 0)),
        _full((8, _HD)),
        _full((_ND, _HD)), _full((_ND, _HD)), _full((_HD, _HD)),
        _full((_ED, _HD)), _full((1, _HD)),
        pl.BlockSpec((_R, 1), lambda i: (i, 0)),
        _full((_HD, _HD // 2)), _full((1, _HD // 2)),
        _full((_HD // 2, 1)), _full((1, 1)),
    ],
    out_specs=[
        _full((_B, 1)),
        _full((_B, _HD)),
    ],
    out_shape=[
        jax.ShapeDtypeStruct((_B, 1), _f32),
        jax.ShapeDtypeStruct((_B, _HD), _f32),
    ],
    scratch_shapes=[
        pltpu.VMEM((_B, _HD), _f32),
        pltpu.VMEM((_B, 1), _f32),
    ],
)


def kernel(x_lex, x_morph, edge_index, batch, prototypes, params):
    p = params
    lex_p = jnp.pad(x_lex.astype(_i32), (0, _NP - _N))
    xmorph_p = jnp.pad(x_morph, ((0, _NP - _N), (0, 0)))
    src_p = jnp.pad(edge_index[0].astype(_i32), (0, _EP - _E))
    dst_p = jnp.pad(edge_index[1].astype(_i32), (0, _EP - _E))
    src2 = src_p.reshape(_EP // 128, 128)
    dst2 = dst_p.reshape(_EP // 128, 128)
    bat_p = jnp.pad(batch.astype(_i32), (0, _NP - _N),
                    constant_values=_B).reshape(_NP, 1)
    p8 = jnp.pad(prototypes, ((0, 3), (0, 0)))
    wb = jnp.stack([p['W_v2'][:, 0],
                    jnp.broadcast_to(p['b_v2'], (_HD,))]).astype(_f32)
    zdeg = jnp.zeros((_NP, _DEGW), _f32)
    zacc = jnp.zeros((_NP, _ED), _f32)
    l0, l1 = p['layers'][0], p['layers'][1]

    x_le = _sc_embed(p['emb'], lex_p)
    x, xm, u, v, t1 = _tc_prep(
        x_le, xmorph_p, p['W_morph'], p['b_morph'][None], p['W_vs'],
        p['W_vd'], p['b_v1'][None], l0['W_nbr'])
    m, deg = _sc_mask(u, v, src2, dst2, wb, zdeg)
    acc1 = _sc_agg(t1, m, src2, dst2, zacc)
    h1, t2 = _tc_layer1(x, acc1, deg, xm, p8, l0['W_self'], l0['W_q'],
                        l0['W_ctx'], l0['W_mrf'], l0['b'][None], l1['W_nbr'])
    acc2 = _sc_agg(t2, m, src2, dst2, zacc)
    logits, ge = _tc_layer2(h1, acc2, deg, xm, p8, l1['W_self'], l1['W_q'],
                            l1['W_ctx'], l1['W_mrf'], l1['b'][None], bat_p,
                            p['W_c1'], p['b_c1'][None], p['W_c2'],
                            p['b_c2'][None])
    return (logits, ge, m[:_E])


# speedup vs baseline: 1.1724x; 1.0096x over previous
import functools

import jax
import jax.numpy as jnp
from jax import lax
from jax.experimental import pallas as pl
from jax.experimental.pallas import tpu as pltpu
from jax.experimental.pallas import tpu_sc as plsc

_N = 50000
_E = 800000
_ED = 32
_HD = 64
_ND = 64
_B = 64
_NC = 2
_NS = 16
_L = 16
_NW = _NC * _NS
_NP = 50176
_EP = 819200
_NPT = _NP // _NW
_EPT = _EP // _NW
_CHUNK = 512
_JROWS = _CHUNK // 128
_NCHUNK = _EPT // _CHUNK
_DEGW = 16
_ACHUNK = 256
_AJROWS = _ACHUNK // 128
_SPROWS = _NP // _NS

_f32 = jnp.float32
_i32 = jnp.int32

_MESH = plsc.VectorSubcoreMesh(core_axis_name="c", subcore_axis_name="s")
_SC_PARAMS = pltpu.CompilerParams(use_tc_tiling_on_sc=False,
                                  needs_layout_passes=False)


def _wid():
    return lax.axis_index("s") * _NC + lax.axis_index("c")


def _sc_embed_body(emb_hbm, lex_hbm, out_hbm, idx_v, rows_v, sem):
    base = _wid() * _NPT
    pltpu.sync_copy(lex_hbm.at[pl.ds(base, _NPT)], idx_v)
    cps = []
    for i in range(14):
        o = i * 112
        cps.append(pltpu.async_copy(
            emb_hbm.at[idx_v.at[pl.ds(o, 112)]], rows_v.at[pl.ds(o, 112)], sem))
    for cp in cps:
        cp.wait()
    pltpu.sync_copy(rows_v, out_hbm.at[pl.ds(base, _NPT)])


_sc_embed = pl.kernel(
    _sc_embed_body,
    out_type=jax.ShapeDtypeStruct((_NP, _ED), _f32),
    mesh=_MESH,
    compiler_params=_SC_PARAMS,
    scratch_types=[
        pltpu.VMEM((_NPT,), _i32),
        pltpu.VMEM((_NPT, _ED), _f32),
        pltpu.SemaphoreType.DMA,
    ],
)


def _sc_mask_body(u_hbm, v_hbm, src_hbm, dst_hbm, wb_hbm, zdeg_hbm,
                  m_hbm, deg_hbm,
                  idx_s, idx_d, ubuf, vbuf, mbuf, dbuf, wbv, z16, deg_sp, sem):
    c = lax.axis_index("c")
    s = lax.axis_index("s")
    w = _wid()
    pltpu.sync_copy(zdeg_hbm.at[pl.ds(s * _SPROWS, _SPROWS)],
                    deg_sp.at[pl.ds(s * _SPROWS, _SPROWS)])
    pltpu.sync_copy(zdeg_hbm.at[pl.ds(0, _CHUNK)], dbuf)
    pltpu.sync_copy(wb_hbm, wbv)
    pltpu.sync_copy(zdeg_hbm.at[0].at[pl.ds(0, _L)], z16)
    plsc.subcore_barrier()

    iota = lax.iota(_i32, _L)
    zero16 = plsc.bitcast(z16[...], _i32)
    one16 = jnp.full((_L,), 1, _i32)
    bv2 = plsc.load_gather(wbv, [one16, zero16])

    def chunk_body(ch, carry):
        ebase = w * _EPT + ch * _CHUNK
        rbase = w * (_EPT // 128) + ch * _JROWS
        pltpu.sync_copy(src_hbm.at[pl.ds(rbase, _JROWS)], idx_s)
        pltpu.sync_copy(dst_hbm.at[pl.ds(rbase, _JROWS)], idx_d)
        cps = []
        for j in range(_JROWS):
            cps.append(pltpu.async_copy(
                u_hbm.at[idx_s.at[j]], ubuf.at[pl.ds(j * 128, 128)], sem))
            cps.append(pltpu.async_copy(
                v_hbm.at[idx_d.at[j]], vbuf.at[pl.ds(j * 128, 128)], sem))
        for cp in cps:
            cp.wait()

        def g_body(g, carry2):
            eidx = iota + g * _L
            accs = [bv2, None, None, None]
            for f in range(_ND):
                f16 = zero16 if f == 0 else jnp.full((_L,), f, _i32)
                uf = plsc.load_gather(ubuf, [eidx, f16])
                vf = plsc.load_gather(vbuf, [eidx, f16])
                w2f = plsc.load_gather(wbv, [zero16, f16])
                t = jnp.maximum(uf + vf, 0.0) * w2f
                k = f % 4
                accs[k] = t if accs[k] is None else accs[k] + t
            acc = (accs[0] + accs[1]) + (accs[2] + accs[3])
            m16 = 1.0 / (1.0 + jnp.exp(-acc))
            geid = ebase + g * _L + iota
            m16 = jnp.where(geid < _E, m16, 0.0)
            mbuf[pl.ds(g * _L, _L)] = m16
            plsc.store_scatter(dbuf, [eidx, zero16], m16)
            return carry2

        lax.fori_loop(0, _CHUNK // _L, g_body, 0, unroll=False)
        pltpu.sync_copy(mbuf, m_hbm.at[pl.ds(ebase, _CHUNK)])
        for j in range(_JROWS):
            pltpu.sync_copy(dbuf.at[pl.ds(j * 128, 128)],
                            deg_sp.at[idx_d.at[j]], add=True)
        return carry

    lax.fori_loop(0, _NCHUNK, chunk_body, 0, unroll=False)
    plsc.subcore_barrier()
    pltpu.sync_copy(deg_sp.at[pl.ds(s * _SPROWS, _SPROWS)],
                    deg_hbm.at[c].at[pl.ds(s * _SPROWS, _SPROWS)])


_sc_mask = pl.kernel(
    _sc_mask_body,
    out_type=(jax.ShapeDtypeStruct((_EP,), _f32),
              jax.ShapeDtypeStruct((_NC, _NP, _DEGW), _f32)),
    mesh=_MESH,
    compiler_params=_SC_PARAMS,
    scratch_types=[
        pltpu.VMEM((_JROWS, 128), _i32),
        pltpu.VMEM((_JROWS, 128), _i32),
        pltpu.VMEM((_CHUNK, _ND), _f32),
        pltpu.VMEM((_CHUNK, _ND), _f32),
        pltpu.VMEM((_CHUNK,), _f32),
        pltpu.VMEM((_CHUNK, _DEGW), _f32),
        pltpu.VMEM((2, _ND), _f32),
        pltpu.VMEM((_L,), _f32),
        pltpu.VMEM_SHARED((_NP, _DEGW), _f32),
        pltpu.SemaphoreType.DMA,
    ],
)


def _sc_agg_body(tab_hbm, m_hbm, src_hbm, dst_hbm, zacc_hbm,
                 acc_out_hbm,
                 idx_s, idx_d, tbuf, obuf, mbuf, z16, acc_sp, sem):
    c = lax.axis_index("c")
    s = lax.axis_index("s")
    pltpu.sync_copy(zacc_hbm.at[pl.ds(s * _SPROWS, _SPROWS)],
                    acc_sp.at[pl.ds(s * _SPROWS, _SPROWS)])
    pltpu.sync_copy(zacc_hbm.at[0].at[pl.ds(0, _L)], z16)
    plsc.subcore_barrier()

    iota = lax.iota(_i32, _L)
    zero16 = plsc.bitcast(z16[...], _i32)
    _EPS = _EP // _NS

    def chunk_body(ch, carry):
        ebase = s * _EPS + ch * _ACHUNK
        rbase = s * (_EPS // 128) + ch * _AJROWS
        pltpu.sync_copy(src_hbm.at[pl.ds(rbase, _AJROWS)], idx_s)
        pltpu.sync_copy(dst_hbm.at[pl.ds(rbase, _AJROWS)], idx_d)
        pltpu.sync_copy(m_hbm.at[pl.ds(ebase, _ACHUNK)], mbuf)
        cps = []
        for j in range(_AJROWS):
            cps.append(pltpu.async_copy(
                tab_hbm.at[c].at[idx_s.at[j]], tbuf.at[pl.ds(j * 128, 128)],
                sem))
        for cp in cps:
            cp.wait()

        def g_body(g, carry2):
            eidx = iota + g * _L
            m16 = mbuf[pl.ds(g * _L, _L)]
            for f in range(_ED):
                f16 = zero16 if f == 0 else jnp.full((_L,), f, _i32)
                col = plsc.load_gather(tbuf, [eidx, f16]) * m16
                plsc.store_scatter(obuf, [eidx, f16], col)
            return carry2

        lax.fori_loop(0, _ACHUNK // _L, g_body, 0, unroll=False)
        for j in range(_AJROWS):
            pltpu.sync_copy(obuf.at[pl.ds(j * 128, 128)],
                            acc_sp.at[idx_d.at[j]], add=True)
        return carry

    lax.fori_loop(0, _EPS // _ACHUNK, chunk_body, 0, unroll=False)
    plsc.subcore_barrier()
    pltpu.sync_copy(acc_sp.at[pl.ds(s * _SPROWS, _SPROWS)],
                    acc_out_hbm.at[c].at[pl.ds(s * _SPROWS, _SPROWS)])


_sc_agg = pl.kernel(
    _sc_agg_body,
    out_type=jax.ShapeDtypeStruct((_NC, _NP, _ED), _f32),
    mesh=_MESH,
    compiler_params=_SC_PARAMS,
    scratch_types=[
        pltpu.VMEM((_AJROWS, 128), _i32),
        pltpu.VMEM((_AJROWS, 128), _i32),
        pltpu.VMEM((_ACHUNK, _ED), _f32),
        pltpu.VMEM((_ACHUNK, _ED), _f32),
        pltpu.VMEM((_ACHUNK,), _f32),
        pltpu.VMEM((_L,), _f32),
        pltpu.VMEM_SHARED((_NP, _ED), _f32),
        pltpu.SemaphoreType.DMA,
    ],
)


_R = 1568
_G = _NP // _R


def _dot(a, b):
    return jnp.dot(a, b, preferred_element_type=_f32)


def _tc_prep_body(xle, xmorph, wm, bm, wvs, wvd, bv1, wnbr,
                  x_out, xm_out, u_out, v_out, t1_out):
    xm = jnp.maximum(_dot(xmorph[...], wm[...]) + bm[...], 0.0)
    x = jnp.concatenate([xle[...], xm], axis=1)
    x_out[...] = x
    xm_out[...] = xm
    u_out[...] = _dot(x, wvs[...]) + bv1[...]
    v_out[...] = _dot(x, wvd[...])
    t1 = _dot(x, wnbr[...])
    t1_out[0] = t1[:, :_ED]
    t1_out[1] = t1[:, _ED:]


def _full(shape):
    return pl.BlockSpec(shape, lambda i: (0,) * len(shape))


_tc_prep = pl.pallas_call(
    _tc_prep_body,
    grid=(_G,),
    in_specs=[
        pl.BlockSpec((_R, _ED), lambda i: (i, 0)),
        pl.BlockSpec((_R, 16), lambda i: (i, 0)),
        _full((16, _ED)), _full((1, _ED)),
        _full((_ND, _HD)), _full((_ND, _HD)), _full((1, _HD)),
        _full((_ND, _HD)),
    ],
    out_specs=[
        pl.BlockSpec((_R, _ND), lambda i: (i, 0)),
        pl.BlockSpec((_R, _ED), lambda i: (i, 0)),
        pl.BlockSpec((_R, _ND), lambda i: (i, 0)),
        pl.BlockSpec((_R, _ND), lambda i: (i, 0)),
        pl.BlockSpec((2, _R, _ED), lambda i: (0, i, 0)),
    ],
    out_shape=[
        jax.ShapeDtypeStruct((_NP, _ND), _f32),
        jax.ShapeDtypeStruct((_NP, _ED), _f32),
        jax.ShapeDtypeStruct((_NP, _ND), _f32),
        jax.ShapeDtypeStruct((_NP, _ND), _f32),
        jax.ShapeDtypeStruct((2, _NP, _ED), _f32),
    ],
)


def _node_update(h, acc, deg2, xm, p8, wself, wq, wctx, wmrf, b):
    deg = deg2[0][:, 0:1] + deg2[1][:, 0:1]
    agg = jnp.concatenate([acc[0], acc[1]], axis=1) / (deg + 1e-6)
    q = _dot(h, wq)
    scores = lax.dot_general(q, p8, (((1,), (1,)), ((), ())),
                             preferred_element_type=_f32)
    col = lax.broadcasted_iota(_i32, (1, 8), 1)
    scores = jnp.where(col < 5, scores, -1e30)
    scores = scores - jnp.max(scores, axis=1, keepdims=True)
    e = jnp.exp(scores)
    alpha = e / jnp.sum(e, axis=1, keepdims=True)
    ctx = _dot(alpha, p8)
    return jnp.maximum(
        _dot(h, wself) + agg + _dot(ctx, wctx) + _dot(xm, wmrf) + b[...], 0.0)


def _tc_layer1_body(h, acc, deg2, xm, p8, wself, wq, wctx, wmrf, b, wnbr2,
                    h1_out, t2_out):
    h1 = _node_update(h[...], acc[...], deg2[...], xm[...], p8[...],
                      wself[...], wq[...], wctx[...], wmrf[...], b)
    h1_out[...] = h1
    t2 = _dot(h1, wnbr2[...])
    t2_out[0] = t2[:, :_ED]
    t2_out[1] = t2[:, _ED:]


_tc_layer1 = pl.pallas_call(
    _tc_layer1_body,
    grid=(_G,),
    in_specs=[
        pl.BlockSpec((_R, _ND), lambda i: (i, 0)),
        pl.BlockSpec((2, _R, _ED), lambda i: (0, i, 0)),
        pl.BlockSpec((2, _R, _DEGW), lambda i: (0, i, 0)),
        pl.BlockSpec((_R, _ED), lambda i: (i, 0)),
        _full((8, _HD)),
        _full((_ND, _HD)), _full((_ND, _HD)), _full((_HD, _HD)),
        _full((_ED, _HD)), _full((1, _HD)),
        _full((_HD, _HD)),
    ],
    out_specs=[
        pl.BlockSpec((_R, _HD), lambda i: (i, 0)),
        pl.BlockSpec((2, _R, _ED), lambda i: (0, i, 0)),
    ],
    out_shape=[
        jax.ShapeDtypeStruct((_NP, _HD), _f32),
        jax.ShapeDtypeStruct((2, _NP, _ED), _f32),
    ],
)


def _tc_layer2_body(h, acc, deg2, xm, p8, wself, wq, wctx, wmrf, b,
                    bat, wc1, bc1, wc2, bc2,
                    logits_out, ge_out, ge_acc, cnt_acc):
    i = pl.program_id(0)
    h2 = _node_update(h[...], acc[...], deg2[...], xm[...], p8[...],
                      wself[...], wq[...], wctx[...], wmrf[...], b)
    onehot = (bat[...] == lax.broadcasted_iota(_i32, (1, _B), 1)).astype(_f32)

    @pl.when(i == 0)
    def _():
        ge_acc[...] = jnp.zeros_like(ge_acc)
        cnt_acc[...] = jnp.zeros_like(cnt_acc)

    ge_acc[...] += lax.dot_general(onehot, h2, (((0,), (0,)), ((), ())),
                                   preferred_element_type=_f32)
    cnt_acc[...] += lax.dot_general(
        onehot, jnp.ones((_R, 1), _f32), (((0,), (0,)), ((), ())),
        preferred_element_type=_f32)

    @pl.when(i == _G - 1)
    def _():
        ge = ge_acc[...] / (cnt_acc[...] + 1e-6)
        hid = jnp.maximum(_dot(ge, wc1[...]) + bc1[...], 0.0)
        logits_out[...] = _dot(hid, wc2[...]) + bc2[...]
        ge_out[...] = ge


_tc_layer2 = pl.pallas_call(
    _tc_layer2_body,
    grid=(_G,),
    in_specs=[
        pl.BlockSpec((_R, _HD), lambda i: (i, 0)),
        pl.BlockSpec((2, _R, _ED), lambda i: (0, i, 0)),
        pl.BlockSpec((2, _R, _DEGW), lambda i: (0, i, 0)),
        pl.BlockSpec((_R, _ED), lambda i: (i, 0)),
        _full((8, _HD)),
        _full((_ND, _HD)), _full((_ND, _HD)), _full((_HD, _HD)),
        _full((_ED, _HD)), _full((1, _HD)),
        pl.BlockSpec((_R, 1), lambda i: (i, 0)),
        _full((_HD, _HD // 2)), _full((1, _HD // 2)),
        _full((_HD // 2, 1)), _full((1, 1)),
    ],
    out_specs=[
        _full((_B, 1)),
        _full((_B, _HD)),
    ],
    out_shape=[
        jax.ShapeDtypeStruct((_B, 1), _f32),
        jax.ShapeDtypeStruct((_B, _HD), _f32),
    ],
    scratch_shapes=[
        pltpu.VMEM((_B, _HD), _f32),
        pltpu.VMEM((_B, 1), _f32),
    ],
)


def kernel(x_lex, x_morph, edge_index, batch, prototypes, params):
    p = params
    lex_p = jnp.pad(x_lex.astype(_i32), (0, _NP - _N))
    xmorph_p = jnp.pad(x_morph, ((0, _NP - _N), (0, 0)))
    src_p = jnp.pad(edge_index[0].astype(_i32), (0, _EP - _E))
    dst_p = jnp.pad(edge_index[1].astype(_i32), (0, _EP - _E))
    src2 = src_p.reshape(_EP // 128, 128)
    dst2 = dst_p.reshape(_EP // 128, 128)
    bat_p = jnp.pad(batch.astype(_i32), (0, _NP - _N),
                    constant_values=_B).reshape(_NP, 1)
    p8 = jnp.pad(prototypes, ((0, 3), (0, 0)))
    wb = jnp.stack([p['W_v2'][:, 0],
                    jnp.broadcast_to(p['b_v2'], (_HD,))]).astype(_f32)
    zdeg = jnp.zeros((_NP, _DEGW), _f32)
    zacc = jnp.zeros((_NP, _ED), _f32)
    l0, l1 = p['layers'][0], p['layers'][1]

    x_le = _sc_embed(p['emb'], lex_p)
    x, xm, u, v, t1 = _tc_prep(
        x_le, xmorph_p, p['W_morph'], p['b_morph'][None], p['W_vs'],
        p['W_vd'], p['b_v1'][None], l0['W_nbr'])
    m, deg = _sc_mask(u, v, src2, dst2, wb, zdeg)
    acc1 = _sc_agg(t1, m, src2, dst2, zacc)
    h1, t2 = _tc_layer1(x, acc1, deg, xm, p8, l0['W_self'], l0['W_q'],
                        l0['W_ctx'], l0['W_mrf'], l0['b'][None], l1['W_nbr'])
    acc2 = _sc_agg(t2, m, src2, dst2, zacc)
    logits, ge = _tc_layer2(h1, acc2, deg, xm, p8, l1['W_self'], l1['W_q'],
                            l1['W_ctx'], l1['W_mrf'], l1['b'][None], bat_p,
                            p['W_c1'], p['b_c1'][None], p['W_c2'],
                            p['b_c2'][None])
    return (logits, ge, m[:_E])


# speedup vs baseline: 1.1903x; 1.0153x over previous
import functools

import jax
import jax.numpy as jnp
from jax import lax
from jax.experimental import pallas as pl
from jax.experimental.pallas import tpu as pltpu
from jax.experimental.pallas import tpu_sc as plsc

_N = 50000
_E = 800000
_ED = 32
_HD = 64
_ND = 64
_B = 64
_NC = 2
_NS = 16
_L = 16
_NW = _NC * _NS
_NP = 50176
_EP = 819200
_NPT = _NP // _NW
_EPT = _EP // _NW
_CHUNK = 512
_JROWS = _CHUNK // 128
_NCHUNK = _EPT // _CHUNK
_DEGW = 8
_ACHUNK = 256
_AJROWS = _ACHUNK // 128
_SPROWS = _NP // _NS

_f32 = jnp.float32
_i32 = jnp.int32

_MESH = plsc.VectorSubcoreMesh(core_axis_name="c", subcore_axis_name="s")
_SC_PARAMS = pltpu.CompilerParams(use_tc_tiling_on_sc=False,
                                  needs_layout_passes=False)


def _wid():
    return lax.axis_index("s") * _NC + lax.axis_index("c")


def _sc_embed_body(emb_hbm, lex_hbm, out_hbm, idx_v, rows_v, sem):
    base = _wid() * _NPT
    pltpu.sync_copy(lex_hbm.at[pl.ds(base, _NPT)], idx_v)
    cps = []
    for i in range(14):
        o = i * 112
        cps.append(pltpu.async_copy(
            emb_hbm.at[idx_v.at[pl.ds(o, 112)]], rows_v.at[pl.ds(o, 112)], sem))
    for cp in cps:
        cp.wait()
    pltpu.sync_copy(rows_v, out_hbm.at[pl.ds(base, _NPT)])


_sc_embed = pl.kernel(
    _sc_embed_body,
    out_type=jax.ShapeDtypeStruct((_NP, _ED), _f32),
    mesh=_MESH,
    compiler_params=_SC_PARAMS,
    scratch_types=[
        pltpu.VMEM((_NPT,), _i32),
        pltpu.VMEM((_NPT, _ED), _f32),
        pltpu.SemaphoreType.DMA,
    ],
)


def _sc_mask_body(u_hbm, v_hbm, src_hbm, dst_hbm, wb_hbm, zdeg_hbm,
                  m_hbm, deg_hbm,
                  idx_s, idxa_d, idxb_d, ubuf, vbuf, mbufa, mbufb,
                  dbufa, dbufb, wbv, z16, deg_sp, semg, sems, semm):
    c = lax.axis_index("c")
    s = lax.axis_index("s")
    w = _wid()
    pltpu.sync_copy(zdeg_hbm.at[pl.ds(s * _SPROWS, _SPROWS)],
                    deg_sp.at[pl.ds(s * _SPROWS, _SPROWS)])
    pltpu.sync_copy(zdeg_hbm.at[pl.ds(0, _CHUNK)], dbufa)
    pltpu.sync_copy(zdeg_hbm.at[pl.ds(0, _CHUNK)], dbufb)
    pltpu.sync_copy(wb_hbm, wbv)
    pltpu.sync_copy(zdeg_hbm.at[0].at[pl.ds(0, _L)], z16)
    plsc.subcore_barrier()

    iota = lax.iota(_i32, _L)
    zero16 = plsc.bitcast(z16[...], _i32)
    one16 = jnp.full((_L,), 1, _i32)
    bv2 = plsc.load_gather(wbv, [one16, zero16])

    def half(it, k, idx_d, mbuf, dbuf):
        ch = it * 2 + k
        ebase = w * _EPT + ch * _CHUNK
        rbase = w * (_EPT // 128) + ch * _JROWS

        @pl.when(it > 0)
        def _():
            for j in range(_JROWS):
                pltpu.make_async_copy(
                    dbuf.at[pl.ds(j * 128, 128)],
                    deg_sp.at[idx_d.at[j]], sems).wait()
            pltpu.make_async_copy(mbuf, m_hbm.at[pl.ds(ebase, _CHUNK)],
                                  semm).wait()

        pltpu.sync_copy(src_hbm.at[pl.ds(rbase, _JROWS)], idx_s)
        pltpu.sync_copy(dst_hbm.at[pl.ds(rbase, _JROWS)], idx_d)
        cps = []
        for j in range(_JROWS):
            cps.append(pltpu.async_copy(
                u_hbm.at[idx_s.at[j]], ubuf.at[pl.ds(j * 128, 128)], semg))
            cps.append(pltpu.async_copy(
                v_hbm.at[idx_d.at[j]], vbuf.at[pl.ds(j * 128, 128)], semg))
        for cp in cps:
            cp.wait()

        def g_body(g, carry2):
            eidx = iota + g * _L
            accs = [bv2, None, None, None]
            for f in range(_ND):
                f16 = zero16 if f == 0 else jnp.full((_L,), f, _i32)
                uf = plsc.load_gather(ubuf, [eidx, f16])
                vf = plsc.load_gather(vbuf, [eidx, f16])
                w2f = plsc.load_gather(wbv, [zero16, f16])
                he = jnp.maximum(uf + vf, 0.0)
                xi = plsc.bitcast(he, _i32)
                xi = (xi + 0x7FFF + ((xi >> 16) & 1)) & (-65536)
                t = plsc.bitcast(xi, _f32) * w2f
                kk = f % 4
                accs[kk] = t if accs[kk] is None else accs[kk] + t
            acc = (accs[0] + accs[1]) + (accs[2] + accs[3])
            m16 = 1.0 / (1.0 + jnp.exp(-acc))
            geid = ebase + g * _L + iota
            m16 = jnp.where(geid < _E, m16, 0.0)
            mbuf[pl.ds(g * _L, _L)] = m16
            plsc.store_scatter(dbuf, [eidx, zero16], m16)
            return carry2

        lax.fori_loop(0, _CHUNK // _L, g_body, 0, unroll=False)
        pltpu.async_copy(mbuf, m_hbm.at[pl.ds(ebase, _CHUNK)], semm)
        for j in range(_JROWS):
            pltpu.async_copy(dbuf.at[pl.ds(j * 128, 128)],
                            deg_sp.at[idx_d.at[j]], sems, add=True)

    def chunk_body(it, carry):
        half(it, 0, idxa_d, mbufa, dbufa)
        half(it, 1, idxb_d, mbufb, dbufb)
        return carry

    n_it = _NCHUNK // 2
    lax.fori_loop(0, n_it, chunk_body, 0, unroll=False)
    for idx_d, mbuf, dbuf, ch in ((idxa_d, mbufa, dbufa, _NCHUNK - 2),
                                  (idxb_d, mbufb, dbufb, _NCHUNK - 1)):
        for j in range(_JROWS):
            pltpu.make_async_copy(dbuf.at[pl.ds(j * 128, 128)],
                                  deg_sp.at[idx_d.at[j]], sems).wait()
        pltpu.make_async_copy(
            mbuf, m_hbm.at[pl.ds(w * _EPT + ch * _CHUNK, _CHUNK)], semm).wait()
    plsc.subcore_barrier()
    pltpu.sync_copy(deg_sp.at[pl.ds(s * _SPROWS, _SPROWS)],
                    deg_hbm.at[c].at[pl.ds(s * _SPROWS, _SPROWS)])


_sc_mask = pl.kernel(
    _sc_mask_body,
    out_type=(jax.ShapeDtypeStruct((_EP,), _f32),
              jax.ShapeDtypeStruct((_NC, _NP, _DEGW), _f32)),
    mesh=_MESH,
    compiler_params=_SC_PARAMS,
    scratch_types=[
        pltpu.VMEM((_JROWS, 128), _i32),
        pltpu.VMEM((_JROWS, 128), _i32),
        pltpu.VMEM((_JROWS, 128), _i32),
        pltpu.VMEM((_CHUNK, _ND), _f32),
        pltpu.VMEM((_CHUNK, _ND), _f32),
        pltpu.VMEM((_CHUNK,), _f32),
        pltpu.VMEM((_CHUNK,), _f32),
        pltpu.VMEM((_CHUNK, _DEGW), _f32),
        pltpu.VMEM((_CHUNK, _DEGW), _f32),
        pltpu.VMEM((2, _ND), _f32),
        pltpu.VMEM((_L,), _f32),
        pltpu.VMEM_SHARED((_NP, _DEGW), _f32),
        pltpu.SemaphoreType.DMA,
        pltpu.SemaphoreType.DMA,
        pltpu.SemaphoreType.DMA,
    ],
)


def _sc_agg_body(tab_hbm, m_hbm, src_hbm, dst_hbm, zacc_hbm,
                 acc_out_hbm,
                 idx_s, idxa_d, idxb_d, tbuf, obufa, obufb, mbuf, z16,
                 acc_sp, semg, sems):
    c = lax.axis_index("c")
    s = lax.axis_index("s")
    pltpu.sync_copy(zacc_hbm.at[pl.ds(s * _SPROWS, _SPROWS)],
                    acc_sp.at[pl.ds(s * _SPROWS, _SPROWS)])
    pltpu.sync_copy(zacc_hbm.at[0].at[pl.ds(0, _L)], z16)
    plsc.subcore_barrier()

    iota = lax.iota(_i32, _L)
    zero16 = plsc.bitcast(z16[...], _i32)
    _EPS = _EP // _NS

    def half(it, k, idx_d, obuf):
        ch = it * 2 + k
        ebase = s * _EPS + ch * _ACHUNK
        rbase = s * (_EPS // 128) + ch * _AJROWS

        @pl.when(it > 0)
        def _():
            for j in range(_AJROWS):
                pltpu.make_async_copy(
                    obuf.at[pl.ds(j * 128, 128)],
                    acc_sp.at[idx_d.at[j]], sems).wait()

        pltpu.sync_copy(src_hbm.at[pl.ds(rbase, _AJROWS)], idx_s)
        pltpu.sync_copy(dst_hbm.at[pl.ds(rbase, _AJROWS)], idx_d)
        pltpu.sync_copy(m_hbm.at[pl.ds(ebase, _ACHUNK)], mbuf)
        cps = []
        for j in range(_AJROWS):
            cps.append(pltpu.async_copy(
                tab_hbm.at[c].at[idx_s.at[j]], tbuf.at[pl.ds(j * 128, 128)],
                semg))
        for cp in cps:
            cp.wait()

        def g_body(g, carry2):
            eidx = iota + g * _L
            m16 = mbuf[pl.ds(g * _L, _L)]
            for f in range(_ED):
                f16 = zero16 if f == 0 else jnp.full((_L,), f, _i32)
                col = plsc.load_gather(tbuf, [eidx, f16]) * m16
                plsc.store_scatter(obuf, [eidx, f16], col)
            return carry2

        lax.fori_loop(0, _ACHUNK // _L, g_body, 0, unroll=False)
        for j in range(_AJROWS):
            pltpu.async_copy(obuf.at[pl.ds(j * 128, 128)],
                             acc_sp.at[idx_d.at[j]], sems, add=True)

    def chunk_body(it, carry):
        half(it, 0, idxa_d, obufa)
        half(it, 1, idxb_d, obufb)
        return carry

    lax.fori_loop(0, _EPS // _ACHUNK // 2, chunk_body, 0, unroll=False)
    for idx_d, obuf in ((idxa_d, obufa), (idxb_d, obufb)):
        for j in range(_AJROWS):
            pltpu.make_async_copy(obuf.at[pl.ds(j * 128, 128)],
                                  acc_sp.at[idx_d.at[j]], sems).wait()
    plsc.subcore_barrier()
    pltpu.sync_copy(acc_sp.at[pl.ds(s * _SPROWS, _SPROWS)],
                    acc_out_hbm.at[c].at[pl.ds(s * _SPROWS, _SPROWS)])


_sc_agg = pl.kernel(
    _sc_agg_body,
    out_type=jax.ShapeDtypeStruct((_NC, _NP, _ED), _f32),
    mesh=_MESH,
    compiler_params=_SC_PARAMS,
    scratch_types=[
        pltpu.VMEM((_AJROWS, 128), _i32),
        pltpu.VMEM((_AJROWS, 128), _i32),
        pltpu.VMEM((_AJROWS, 128), _i32),
        pltpu.VMEM((_ACHUNK, _ED), _f32),
        pltpu.VMEM((_ACHUNK, _ED), _f32),
        pltpu.VMEM((_ACHUNK, _ED), _f32),
        pltpu.VMEM((_ACHUNK,), _f32),
        pltpu.VMEM((_L,), _f32),
        pltpu.VMEM_SHARED((_NP, _ED), _f32),
        pltpu.SemaphoreType.DMA,
        pltpu.SemaphoreType.DMA,
    ],
)


_R = 1568
_G = _NP // _R


def _dot(a, b):
    return jnp.dot(a, b, preferred_element_type=_f32)


def _tc_prep_body(xle, xmorph, wm, bm, wvs, wvd, bv1, wnbr,
                  x_out, xm_out, u_out, v_out, t1_out):
    xm = jnp.maximum(_dot(xmorph[...], wm[...]) + bm[...], 0.0)
    x = jnp.concatenate([xle[...], xm], axis=1)
    x_out[...] = x
    xm_out[...] = xm
    u_out[...] = _dot(x, wvs[...]) + bv1[...]
    v_out[...] = _dot(x, wvd[...])
    t1 = _dot(x, wnbr[...])
    t1_out[0] = t1[:, :_ED]
    t1_out[1] = t1[:, _ED:]


def _full(shape):
    return pl.BlockSpec(shape, lambda i: (0,) * len(shape))


_tc_prep = pl.pallas_call(
    _tc_prep_body,
    grid=(_G,),
    in_specs=[
        pl.BlockSpec((_R, _ED), lambda i: (i, 0)),
        pl.BlockSpec((_R, 16), lambda i: (i, 0)),
        _full((16, _ED)), _full((1, _ED)),
        _full((_ND, _HD)), _full((_ND, _HD)), _full((1, _HD)),
        _full((_ND, _HD)),
    ],
    out_specs=[
        pl.BlockSpec((_R, _ND), lambda i: (i, 0)),
        pl.BlockSpec((_R, _ED), lambda i: (i, 0)),
        pl.BlockSpec((_R, _ND), lambda i: (i, 0)),
        pl.BlockSpec((_R, _ND), lambda i: (i, 0)),
        pl.BlockSpec((2, _R, _ED), lambda i: (0, i, 0)),
    ],
    out_shape=[
        jax.ShapeDtypeStruct((_NP, _ND), _f32),
        jax.ShapeDtypeStruct((_NP, _ED), _f32),
        jax.ShapeDtypeStruct((_NP, _ND), _f32),
        jax.ShapeDtypeStruct((_NP, _ND), _f32),
        jax.ShapeDtypeStruct((2, _NP, _ED), _f32),
    ],
)


def _node_update(h, acc, deg2, xm, p8, wself, wq, wctx, wmrf, b):
    deg = deg2[0][:, 0:1] + deg2[1][:, 0:1]
    agg = jnp.concatenate([acc[0], acc[1]], axis=1) / (deg + 1e-6)
    q = _dot(h, wq)
    scores = lax.dot_general(q, p8, (((1,), (1,)), ((), ())),
                             preferred_element_type=_f32)
    col = lax.broadcasted_iota(_i32, (1, 8), 1)
    scores = jnp.where(col < 5, scores, -1e30)
    scores = scores - jnp.max(scores, axis=1, keepdims=True)
    e = jnp.exp(scores)
    alpha = e / jnp.sum(e, axis=1, keepdims=True)
    ctx = _dot(alpha, p8)
    return jnp.maximum(
        _dot(h, wself) + agg + _dot(ctx, wctx) + _dot(xm, wmrf) + b[...], 0.0)


def _tc_layer1_body(h, acc, deg2, xm, p8, wself, wq, wctx, wmrf, b, wnbr2,
                    h1_out, t2_out):
    h1 = _node_update(h[...], acc[...], deg2[...], xm[...], p8[...],
                      wself[...], wq[...], wctx[...], wmrf[...], b)
    h1_out[...] = h1
    t2 = _dot(h1, wnbr2[...])
    t2_out[0] = t2[:, :_ED]
    t2_out[1] = t2[:, _ED:]


_tc_layer1 = pl.pallas_call(
    _tc_layer1_body,
    grid=(_G,),
    in_specs=[
        pl.BlockSpec((_R, _ND), lambda i: (i, 0)),
        pl.BlockSpec((2, _R, _ED), lambda i: (0, i, 0)),
        pl.BlockSpec((2, _R, _DEGW), lambda i: (0, i, 0)),
        pl.BlockSpec((_R, _ED), lambda i: (i, 0)),
        _full((8, _HD)),
        _full((_ND, _HD)), _full((_ND, _HD)), _full((_HD, _HD)),
        _full((_ED, _HD)), _full((1, _HD)),
        _full((_HD, _HD)),
    ],
    out_specs=[
        pl.BlockSpec((_R, _HD), lambda i: (i, 0)),
        pl.BlockSpec((2, _R, _ED), lambda i: (0, i, 0)),
    ],
    out_shape=[
        jax.ShapeDtypeStruct((_NP, _HD), _f32),
        jax.ShapeDtypeStruct((2, _NP, _ED), _f32),
    ],
)


def _tc_layer2_body(h, acc, deg2, xm, p8, wself, wq, wctx, wmrf, b,
                    bat, wc1, bc1, wc2, bc2,
                    logits_out, ge_out, ge_acc, cnt_acc):
    i = pl.program_id(0)
    h2 = _node_update(h[...], acc[...], deg2[...], xm[...], p8[...],
                      wself[...], wq[...], wctx[...], wmrf[...], b)
    onehot = (bat[...] == lax.broadcasted_iota(_i32, (1, _B), 1)).astype(_f32)

    @pl.when(i == 0)
    def _():
        ge_acc[...] = jnp.zeros_like(ge_acc)
        cnt_acc[...] = jnp.zeros_like(cnt_acc)

    ge_acc[...] += lax.dot_general(onehot, h2, (((0,), (0,)), ((), ())),
                                   preferred_element_type=_f32)
    cnt_acc[...] += lax.dot_general(
        onehot, jnp.ones((_R, 1), _f32), (((0,), (0,)), ((), ())),
        preferred_element_type=_f32)

    @pl.when(i == _G - 1)
    def _():
        ge = ge_acc[...] / (cnt_acc[...] + 1e-6)
        hid = jnp.maximum(_dot(ge, wc1[...]) + bc1[...], 0.0)
        logits_out[...] = _dot(hid, wc2[...]) + bc2[...]
        ge_out[...] = ge


_tc_layer2 = pl.pallas_call(
    _tc_layer2_body,
    grid=(_G,),
    in_specs=[
        pl.BlockSpec((_R, _HD), lambda i: (i, 0)),
        pl.BlockSpec((2, _R, _ED), lambda i: (0, i, 0)),
        pl.BlockSpec((2, _R, _DEGW), lambda i: (0, i, 0)),
        pl.BlockSpec((_R, _ED), lambda i: (i, 0)),
        _full((8, _HD)),
        _full((_ND, _HD)), _full((_ND, _HD)), _full((_HD, _HD)),
        _full((_ED, _HD)), _full((1, _HD)),
        pl.BlockSpec((_R, 1), lambda i: (i, 0)),
        _full((_HD, _HD // 2)), _full((1, _HD // 2)),
        _full((_HD // 2, 1)), _full((1, 1)),
    ],
    out_specs=[
        _full((_B, 1)),
        _full((_B, _HD)),
    ],
    out_shape=[
        jax.ShapeDtypeStruct((_B, 1), _f32),
        jax.ShapeDtypeStruct((_B, _HD), _f32),
    ],
    scratch_shapes=[
        pltpu.VMEM((_B, _HD), _f32),
        pltpu.VMEM((_B, 1), _f32),
    ],
)


def kernel(x_lex, x_morph, edge_index, batch, prototypes, params):
    p = params
    lex_p = jnp.pad(x_lex.astype(_i32), (0, _NP - _N))
    xmorph_p = jnp.pad(x_morph, ((0, _NP - _N), (0, 0)))
    src_p = jnp.pad(edge_index[0].astype(_i32), (0, _EP - _E))
    dst_p = jnp.pad(edge_index[1].astype(_i32), (0, _EP - _E))
    src2 = src_p.reshape(_EP // 128, 128)
    dst2 = dst_p.reshape(_EP // 128, 128)
    bat_p = jnp.pad(batch.astype(_i32), (0, _NP - _N),
                    constant_values=_B).reshape(_NP, 1)
    p8 = jnp.pad(prototypes, ((0, 3), (0, 0)))
    w2bf = p['W_v2'][:, 0].astype(jnp.bfloat16).astype(_f32)
    wb = jnp.stack([w2bf,
                    jnp.broadcast_to(p['b_v2'], (_HD,))]).astype(_f32)
    zdeg = jnp.zeros((_NP, _DEGW), _f32)
    zacc = jnp.zeros((_NP, _ED), _f32)
    l0, l1 = p['layers'][0], p['layers'][1]

    x_le = _sc_embed(p['emb'], lex_p)
    x, xm, u, v, t1 = _tc_prep(
        x_le, xmorph_p, p['W_morph'], p['b_morph'][None], p['W_vs'],
        p['W_vd'], p['b_v1'][None], l0['W_nbr'])
    m, deg = _sc_mask(u, v, src2, dst2, wb, zdeg)
    acc1 = _sc_agg(t1, m, src2, dst2, zacc)
    h1, t2 = _tc_layer1(x, acc1, deg, xm, p8, l0['W_self'], l0['W_q'],
                        l0['W_ctx'], l0['W_mrf'], l0['b'][None], l1['W_nbr'])
    acc2 = _sc_agg(t2, m, src2, dst2, zacc)
    logits, ge = _tc_layer2(h1, acc2, deg, xm, p8, l1['W_self'], l1['W_q'],
                            l1['W_ctx'], l1['W_mrf'], l1['b'][None], bat_p,
                            p['W_c1'], p['b_c1'][None], p['W_c2'],
                            p['b_c2'][None])
    return (logits, ge, m[:_E])
